# baseline (speedup 1.0000x reference)
import functools

import jax
import jax.numpy as jnp
from jax.experimental import pallas as pl
from jax.experimental.pallas import tpu as pltpu


def _mlp_kernel(x_ref, w1_ref, b1_ref, w2_ref, b2_ref, o_ref):
    x = x_ref[...].astype(jnp.bfloat16)
    h = jax.lax.dot_general(
        x, w1_ref[...], (((1,), (1,)), ((), ())),
        preferred_element_type=jnp.float32,
    )
    h = jnp.maximum(h + b1_ref[...], 0.0).astype(jnp.bfloat16)
    o = jax.lax.dot_general(
        h, w2_ref[...], (((1,), (1,)), ((), ())),
        preferred_element_type=jnp.float32,
    )
    o_ref[...] = o + b2_ref[...]


@functools.partial(jax.jit, static_argnames=("block_m",))
def _fused_mlp(x, W1, b1, W2, b2, block_m):
    n, d_in = x.shape
    d_hid = W1.shape[0]
    grid = (pl.cdiv(n, block_m),)
    return pl.pallas_call(
        _mlp_kernel,
        grid=grid,
        in_specs=[
            pl.BlockSpec((block_m, d_in), lambda i: (i, 0)),
            pl.BlockSpec((d_hid, d_in), lambda i: (0, 0)),
            pl.BlockSpec((1, d_hid), lambda i: (0, 0)),
            pl.BlockSpec((d_hid, d_hid), lambda i: (0, 0)),
            pl.BlockSpec((1, d_hid), lambda i: (0, 0)),
        ],
        out_specs=pl.BlockSpec((block_m, d_hid), lambda i: (i, 0)),
        out_shape=jax.ShapeDtypeStruct((n, d_hid), jnp.float32),
        compiler_params=pltpu.CompilerParams(
            dimension_semantics=("parallel",),
            vmem_limit_bytes=120 * 1024 * 1024,
        ),
    )(x, W1.astype(jnp.bfloat16), b1.reshape(1, -1),
      W2.astype(jnp.bfloat16), b2.reshape(1, -1))


def kernel(x, W1, b1, W2, b2):
    return _fused_mlp(x, W1, b1, W2, b2, block_m=5000)


# speedup vs baseline: 1.0009x; 1.0009x over previous
import functools

import jax
import jax.numpy as jnp
from jax.experimental import pallas as pl
from jax.experimental.pallas import tpu as pltpu


def _mlp_kernel(x_ref, w1_ref, b1_ref, w2_ref, b2_ref, o_ref):
    x = x_ref[...].astype(jnp.bfloat16)
    h = jax.lax.dot_general(
        x, w1_ref[...], (((1,), (1,)), ((), ())),
        preferred_element_type=jnp.float32,
    )
    h = jnp.maximum(h + b1_ref[...], 0.0).astype(jnp.bfloat16)
    o = jax.lax.dot_general(
        h, w2_ref[...], (((1,), (1,)), ((), ())),
        preferred_element_type=jnp.float32,
    )
    o_ref[...] = o + b2_ref[...]


@functools.partial(jax.jit, static_argnames=("block_m",))
def _fused_mlp(x, W1, b1, W2, b2, block_m):
    n, d_in = x.shape
    d_hid = W1.shape[0]
    grid = (pl.cdiv(n, block_m),)
    return pl.pallas_call(
        _mlp_kernel,
        grid=grid,
        in_specs=[
            pl.BlockSpec((block_m, d_in), lambda i: (i, 0)),
            pl.BlockSpec((d_hid, d_in), lambda i: (0, 0)),
            pl.BlockSpec((1, d_hid), lambda i: (0, 0)),
            pl.BlockSpec((d_hid, d_hid), lambda i: (0, 0)),
            pl.BlockSpec((1, d_hid), lambda i: (0, 0)),
        ],
        out_specs=pl.BlockSpec((block_m, d_hid), lambda i: (i, 0)),
        out_shape=jax.ShapeDtypeStruct((n, d_hid), jnp.float32),
        compiler_params=pltpu.CompilerParams(
            dimension_semantics=("arbitrary",),
            vmem_limit_bytes=120 * 1024 * 1024,
        ),
    )(x, W1.astype(jnp.bfloat16), b1.reshape(1, -1),
      W2.astype(jnp.bfloat16), b2.reshape(1, -1))


def kernel(x, W1, b1, W2, b2):
    return _fused_mlp(x, W1, b1, W2, b2, block_m=5000)


# speedup vs baseline: 1.0322x; 1.0313x over previous
import functools

import jax
import jax.numpy as jnp
from jax.experimental import pallas as pl
from jax.experimental.pallas import tpu as pltpu


def _mlp_kernel(x_ref, w1_ref, b1_ref, w2_ref, b2_ref, o_ref):
    x = x_ref[...]
    h = jax.lax.dot_general(
        x, w1_ref[...], (((1,), (1,)), ((), ())),
        preferred_element_type=jnp.float32,
    )
    h = jnp.maximum(h + b1_ref[...], 0.0)
    o = jax.lax.dot_general(
        h, w2_ref[...], (((1,), (1,)), ((), ())),
        preferred_element_type=jnp.float32,
    )
    o_ref[...] = o + b2_ref[...]


@functools.partial(jax.jit, static_argnames=("block_m",))
def _fused_mlp(x, W1, b1, W2, b2, block_m):
    n, d_in = x.shape
    d_hid = W1.shape[0]
    grid = (pl.cdiv(n, block_m),)
    return pl.pallas_call(
        _mlp_kernel,
        grid=grid,
        in_specs=[
            pl.BlockSpec((block_m, d_in), lambda i: (i, 0)),
            pl.BlockSpec((d_hid, d_in), lambda i: (0, 0)),
            pl.BlockSpec((1, d_hid), lambda i: (0, 0)),
            pl.BlockSpec((d_hid, d_hid), lambda i: (0, 0)),
            pl.BlockSpec((1, d_hid), lambda i: (0, 0)),
        ],
        out_specs=pl.BlockSpec((block_m, d_hid), lambda i: (i, 0)),
        out_shape=jax.ShapeDtypeStruct((n, d_hid), jnp.float32),
        compiler_params=pltpu.CompilerParams(
            dimension_semantics=("arbitrary",),
            vmem_limit_bytes=120 * 1024 * 1024,
        ),
    )(x, W1, b1.reshape(1, -1), W2, b2.reshape(1, -1))


def kernel(x, W1, b1, W2, b2):
    return _fused_mlp(x, W1, b1, W2, b2, block_m=5000)
